# baseline (speedup 1.0000x reference)
import functools

import jax
import jax.numpy as jnp
from jax import lax
from jax.experimental import pallas as pl
from jax.experimental.pallas import tpu as pltpu
from jax.experimental.pallas import tpu_sc as plsc

E = 320000
D = 128
G = 512
NC = 2
NS = 16
NW = NC * NS
L = 16
CH = 80
ROWS_PW = E // NW
CPT = ROWS_PW // CH
SEG_PT = G // NS
IDR = E // D
HB = 4

_mesh = plsc.VectorSubcoreMesh(core_axis_name="c", subcore_axis_name="s")


@functools.partial(
    pl.kernel,
    out_type=jax.ShapeDtypeStruct((NC, G, D), jnp.float32),
    mesh=_mesh,
    scratch_types=dict(
        ids_v=pltpu.VMEM((CPT, CH), jnp.int32),
        rows_v=pltpu.VMEM((2, CH, D), jnp.float32),
        sems=pltpu.SemaphoreType.DMA((2,)),
        zseg_v=pltpu.VMEM((SEG_PT, D), jnp.float32),
        facc_v=pltpu.VMEM((SEG_PT, D), jnp.float32),
        acc_sh=pltpu.VMEM_SHARED((G, D), jnp.float32),
    ),
)
def _seg_sum(feat_hbm, ids_hbm, sums_hbm, *, ids_v, rows_v, sems, zseg_v,
             facc_v, acc_sh):
    c = lax.axis_index("c")
    s = lax.axis_index("s")
    w = c * NS + s

    zeros16 = jnp.zeros((L,), jnp.float32)
    for i in range(SEG_PT):
        for j in range(D // L):
            zseg_v[i, pl.ds(j * L, L)] = zeros16

    pltpu.sync_copy(zseg_v, acc_sh.at[pl.ds(s * SEG_PT, SEG_PT)])
    plsc.subcore_barrier()

    pltpu.sync_copy(ids_hbm.at[w], ids_v)

    row0 = w * ROWS_PW

    def _perm(i):
        return lax.rem(i * 37, CPT)

    def _feat_chunk(p):
        return feat_hbm.at[pl.ds(row0 + p * CH, CH)]

    def _step(i, cur, cur_sem, nxt, nxt_sem):
        pltpu.make_async_copy(_feat_chunk(_perm(i)), cur, cur_sem).wait()

        @pl.when(i + 1 < CPT)
        def _():
            pltpu.async_copy(_feat_chunk(_perm(i + 1)), nxt, nxt_sem)

        pltpu.sync_copy(cur, acc_sh.at[ids_v.at[_perm(i)]], add=True)

    pltpu.async_copy(_feat_chunk(0), rows_v.at[0], sems.at[0])

    def chunk(i, carry):
        @pl.when(i % 2 == 0)
        def _():
            _step(i, rows_v.at[0], sems.at[0], rows_v.at[1], sems.at[1])

        @pl.when(i % 2 == 1)
        def _():
            _step(i, rows_v.at[1], sems.at[1], rows_v.at[0], sems.at[0])

        return carry

    lax.fori_loop(0, CPT, chunk, 0)
    plsc.subcore_barrier()

    g0 = s * SEG_PT
    pltpu.sync_copy(acc_sh.at[pl.ds(g0, SEG_PT)], facc_v)
    pltpu.sync_copy(facc_v, sums_hbm.at[c, pl.ds(g0, SEG_PT)])


def _finalize_body(sums_ref, ids_ref, out_ref):
    hi_col = lax.broadcasted_iota(jnp.int32, (G // L, 1), 0)
    lo_col = lax.broadcasted_iota(jnp.int32, (L, 1), 0)

    def blk(r, count2d):
        for q in range(HB):
            row = ids_ref[pl.ds(r * HB + q, 1), :]
            hcmp = (hi_col == (row >> 4)).astype(jnp.float32)
            lcmp = (lo_col == (row & 15)).astype(jnp.float32)
            count2d = count2d + jax.lax.dot_general(
                hcmp, lcmp, (((1,), (1,)), ((), ())),
                preferred_element_type=jnp.float32)
        return count2d

    count2d = lax.fori_loop(0, IDR // HB, blk,
                            jnp.zeros((G // L, L), jnp.float32))

    g_hi = lax.broadcasted_iota(jnp.int32, (G, G // L), 0) >> 4
    e_hi = (g_hi == lax.broadcasted_iota(jnp.int32, (G, G // L), 1))
    tmp = jax.lax.dot_general(
        e_hi.astype(jnp.float32), count2d, (((1,), (0,)), ((), ())),
        preferred_element_type=jnp.float32)
    g_lo = lax.broadcasted_iota(jnp.int32, (G, L), 0) & 15
    m_lo = (g_lo == lax.broadcasted_iota(jnp.int32, (G, L), 1)).astype(
        jnp.float32)
    cnt = jnp.sum(tmp * m_lo, axis=1, keepdims=True)

    total = sums_ref[0] + sums_ref[1]
    out_ref[...] = total / jnp.maximum(cnt, 1.0)


_finalize = pl.pallas_call(
    _finalize_body,
    out_shape=jax.ShapeDtypeStruct((G, D), jnp.float32),
)


def kernel(feat, segment_ids, num_graphs):
    del num_graphs
    ids = segment_ids.astype(jnp.int32)
    sums = _seg_sum(feat, ids.reshape(NW, CPT, CH))
    return _finalize(sums, ids.reshape(IDR, D))


# speedup vs baseline: 1.5148x; 1.5148x over previous
import functools

import jax
import jax.numpy as jnp
from jax import lax
from jax.experimental import pallas as pl
from jax.experimental.pallas import tpu as pltpu
from jax.experimental.pallas import tpu_sc as plsc

E = 320000
D = 128
G = 512
NC = 2
NS = 16
NW = NC * NS
L = 16
CH = 80
ROWS_PW = E // NW
CPT = ROWS_PW // CH
SEG_PT = G // NS
IDR = E // D
HB = 4

_mesh = plsc.VectorSubcoreMesh(core_axis_name="c", subcore_axis_name="s")


@functools.partial(
    pl.kernel,
    out_type=jax.ShapeDtypeStruct((NC, G, D), jnp.float32),
    mesh=_mesh,
    scratch_types=dict(
        ids_v=pltpu.VMEM((CPT, CH), jnp.int32),
        rows_v=pltpu.VMEM((2, CH, D), jnp.float32),
        sems=pltpu.SemaphoreType.DMA((2,)),
        scat_sems=pltpu.SemaphoreType.DMA((2,)),
        zseg_v=pltpu.VMEM((SEG_PT, D), jnp.float32),
        facc_v=pltpu.VMEM((SEG_PT, D), jnp.float32),
        acc_sh=pltpu.VMEM_SHARED((G, D), jnp.float32),
    ),
)
def _seg_sum(feat_hbm, ids_hbm, sums_hbm, *, ids_v, rows_v, sems, scat_sems,
             zseg_v, facc_v, acc_sh):
    c = lax.axis_index("c")
    s = lax.axis_index("s")
    w = c * NS + s

    zeros16 = jnp.zeros((L,), jnp.float32)
    for i in range(SEG_PT):
        for j in range(D // L):
            zseg_v[i, pl.ds(j * L, L)] = zeros16

    pltpu.sync_copy(zseg_v, acc_sh.at[pl.ds(s * SEG_PT, SEG_PT)])
    plsc.subcore_barrier()

    pltpu.sync_copy(ids_hbm.at[w], ids_v)

    row0 = w * ROWS_PW

    def _perm(i):
        return lax.rem(i * 37, CPT)

    def _feat_chunk(p):
        return feat_hbm.at[pl.ds(row0 + p * CH, CH)]

    def _scat_dst(i):
        return acc_sh.at[ids_v.at[_perm(i)]]

    def _step(i, cur, cur_sem, cur_ssem, nxt, nxt_sem, nxt_ssem):
        pltpu.make_async_copy(_feat_chunk(_perm(i)), cur, cur_sem).wait()

        @pl.when(i >= 1)
        def _():
            pltpu.make_async_copy(nxt, _scat_dst(i - 1), nxt_ssem).wait()

        @pl.when(i + 1 < CPT)
        def _():
            pltpu.async_copy(_feat_chunk(_perm(i + 1)), nxt, nxt_sem)

        pltpu.async_copy(cur, _scat_dst(i), cur_ssem, add=True)

    pltpu.async_copy(_feat_chunk(0), rows_v.at[0], sems.at[0])

    def chunk(i, carry):
        @pl.when(i % 2 == 0)
        def _():
            _step(i, rows_v.at[0], sems.at[0], scat_sems.at[0],
                  rows_v.at[1], sems.at[1], scat_sems.at[1])

        @pl.when(i % 2 == 1)
        def _():
            _step(i, rows_v.at[1], sems.at[1], scat_sems.at[1],
                  rows_v.at[0], sems.at[0], scat_sems.at[0])

        return carry

    lax.fori_loop(0, CPT, chunk, 0)
    pltpu.make_async_copy(rows_v.at[0], _scat_dst(CPT - 1),
                          scat_sems.at[0]).wait()
    plsc.subcore_barrier()

    g0 = s * SEG_PT
    pltpu.sync_copy(acc_sh.at[pl.ds(g0, SEG_PT)], facc_v)
    pltpu.sync_copy(facc_v, sums_hbm.at[c, pl.ds(g0, SEG_PT)])


def _hist_body(ids_ref, acc_ref):
    gcol = lax.broadcasted_iota(jnp.int32, (G, 1), 0)

    def blk(r, acc):
        rows = ids_ref[pl.ds(r * HB, HB), :]
        for q in range(HB):
            acc = acc + (gcol == rows[q:q + 1, :]).astype(jnp.float32)
        return acc

    acc_ref[...] = lax.fori_loop(0, IDR // HB, blk,
                                 jnp.zeros((G, D), jnp.float32))


_hist = pl.pallas_call(
    _hist_body,
    out_shape=jax.ShapeDtypeStruct((G, D), jnp.float32),
)


def _finalize_body(sums_ref, acc_ref, out_ref):
    total = sums_ref[0] + sums_ref[1]
    cnt = jnp.sum(acc_ref[...], axis=1, keepdims=True)
    out_ref[...] = total / jnp.maximum(cnt, 1.0)


_finalize = pl.pallas_call(
    _finalize_body,
    out_shape=jax.ShapeDtypeStruct((G, D), jnp.float32),
)


def kernel(feat, segment_ids, num_graphs):
    del num_graphs
    ids = segment_ids.astype(jnp.int32)
    sums = _seg_sum(feat, ids.reshape(NW, CPT, CH))
    acc = _hist(ids.reshape(IDR, D))
    return _finalize(sums, acc)


# speedup vs baseline: 1.5192x; 1.0029x over previous
import functools

import jax
import jax.numpy as jnp
from jax import lax
from jax.experimental import pallas as pl
from jax.experimental.pallas import tpu as pltpu
from jax.experimental.pallas import tpu_sc as plsc

E = 320000
D = 128
G = 512
NC = 2
NS = 16
NW = NC * NS
L = 16
CH = 80
ROWS_PW = E // NW
CPT = ROWS_PW // CH
SEG_PT = G // NS
IDR = E // D
HB = 20

_mesh = plsc.VectorSubcoreMesh(core_axis_name="c", subcore_axis_name="s")


@functools.partial(
    pl.kernel,
    out_type=jax.ShapeDtypeStruct((NC, G, D), jnp.float32),
    mesh=_mesh,
    scratch_types=dict(
        ids_v=pltpu.VMEM((CPT, CH), jnp.int32),
        rows_v=pltpu.VMEM((2, CH, D), jnp.float32),
        sems=pltpu.SemaphoreType.DMA((2,)),
        zseg_v=pltpu.VMEM((SEG_PT, D), jnp.float32),
        facc_v=pltpu.VMEM((SEG_PT, D), jnp.float32),
        acc_sh=pltpu.VMEM_SHARED((G, D), jnp.float32),
    ),
)
def _seg_sum(feat_hbm, ids_hbm, sums_hbm, *, ids_v, rows_v, sems, zseg_v,
             facc_v, acc_sh):
    c = lax.axis_index("c")
    s = lax.axis_index("s")
    w = c * NS + s

    zeros16 = jnp.zeros((L,), jnp.float32)
    for i in range(SEG_PT):
        for j in range(D // L):
            zseg_v[i, pl.ds(j * L, L)] = zeros16

    pltpu.sync_copy(zseg_v, acc_sh.at[pl.ds(s * SEG_PT, SEG_PT)])
    plsc.subcore_barrier()

    pltpu.sync_copy(ids_hbm.at[w], ids_v)

    row0 = w * ROWS_PW

    def _perm(i):
        return lax.rem(i * 37, CPT)

    def _feat_chunk(p):
        return feat_hbm.at[pl.ds(row0 + p * CH, CH)]

    def _step(i, cur, cur_sem, nxt, nxt_sem):
        pltpu.make_async_copy(_feat_chunk(_perm(i)), cur, cur_sem).wait()

        @pl.when(i + 1 < CPT)
        def _():
            pltpu.async_copy(_feat_chunk(_perm(i + 1)), nxt, nxt_sem)

        pltpu.sync_copy(cur, acc_sh.at[ids_v.at[_perm(i)]], add=True)

    pltpu.async_copy(_feat_chunk(0), rows_v.at[0], sems.at[0])

    def chunk(i, carry):
        @pl.when(i % 2 == 0)
        def _():
            _step(i, rows_v.at[0], sems.at[0], rows_v.at[1], sems.at[1])

        @pl.when(i % 2 == 1)
        def _():
            _step(i, rows_v.at[1], sems.at[1], rows_v.at[0], sems.at[0])

        return carry

    lax.fori_loop(0, CPT, chunk, 0)
    plsc.subcore_barrier()

    g0 = s * SEG_PT
    pltpu.sync_copy(acc_sh.at[pl.ds(g0, SEG_PT)], facc_v)
    pltpu.sync_copy(facc_v, sums_hbm.at[c, pl.ds(g0, SEG_PT)])


def _hist_body(ids_ref, acc_ref):
    gcol = lax.broadcasted_iota(jnp.int32, (G, 1), 0)

    def blk(r, acc):
        rows = ids_ref[pl.ds(r * HB, HB), :]
        for q in range(HB):
            acc = acc + (gcol == rows[q:q + 1, :]).astype(jnp.float32)
        return acc

    acc_ref[...] = lax.fori_loop(0, IDR // HB, blk,
                                 jnp.zeros((G, D), jnp.float32))


_hist = pl.pallas_call(
    _hist_body,
    out_shape=jax.ShapeDtypeStruct((G, D), jnp.float32),
)


def _finalize_body(sums_ref, acc_ref, out_ref):
    total = sums_ref[0] + sums_ref[1]
    cnt = jnp.sum(acc_ref[...], axis=1, keepdims=True)
    out_ref[...] = total / jnp.maximum(cnt, 1.0)


_finalize = pl.pallas_call(
    _finalize_body,
    out_shape=jax.ShapeDtypeStruct((G, D), jnp.float32),
)


def kernel(feat, segment_ids, num_graphs):
    del num_graphs
    ids = segment_ids.astype(jnp.int32)
    sums = _seg_sum(feat, ids.reshape(NW, CPT, CH))
    acc = _hist(ids.reshape(IDR, D))
    return _finalize(sums, acc)


# speedup vs baseline: 1.5214x; 1.0014x over previous
import functools

import jax
import jax.numpy as jnp
from jax import lax
from jax.experimental import pallas as pl
from jax.experimental.pallas import tpu as pltpu
from jax.experimental.pallas import tpu_sc as plsc

E = 320000
D = 128
G = 512
NC = 2
NS = 16
NW = NC * NS
L = 16
CH = 80
ROWS_PW = E // NW
CPT = ROWS_PW // CH
SEG_PT = G // NS
IDR = E // D
HB = 4

_mesh = plsc.VectorSubcoreMesh(core_axis_name="c", subcore_axis_name="s")


@functools.partial(
    pl.kernel,
    out_type=jax.ShapeDtypeStruct((NC, G, D), jnp.float32),
    mesh=_mesh,
    scratch_types=dict(
        ids_v=pltpu.VMEM((CPT, CH), jnp.int32),
        rows_v=pltpu.VMEM((2, CH, D), jnp.float32),
        sems=pltpu.SemaphoreType.DMA((2,)),
        zseg_v=pltpu.VMEM((SEG_PT, D), jnp.float32),
        facc_v=pltpu.VMEM((SEG_PT, D), jnp.float32),
        acc_sh=pltpu.VMEM_SHARED((G, D), jnp.float32),
    ),
)
def _seg_sum(feat_hbm, ids_hbm, sums_hbm, *, ids_v, rows_v, sems, zseg_v,
             facc_v, acc_sh):
    c = lax.axis_index("c")
    s = lax.axis_index("s")
    w = c * NS + s

    zeros16 = jnp.zeros((L,), jnp.float32)
    for i in range(SEG_PT):
        for j in range(D // L):
            zseg_v[i, pl.ds(j * L, L)] = zeros16

    pltpu.sync_copy(zseg_v, acc_sh.at[pl.ds(s * SEG_PT, SEG_PT)])
    plsc.subcore_barrier()

    pltpu.sync_copy(ids_hbm.at[w], ids_v)

    row0 = w * ROWS_PW

    def _perm(i):
        return lax.rem(i * 37, CPT)

    def _feat_chunk(p):
        return feat_hbm.at[pl.ds(row0 + p * CH, CH)]

    def _step(i, cur, cur_sem, nxt, nxt_sem):
        pltpu.make_async_copy(_feat_chunk(_perm(i)), cur, cur_sem).wait()

        @pl.when(i + 1 < CPT)
        def _():
            pltpu.async_copy(_feat_chunk(_perm(i + 1)), nxt, nxt_sem)

        pltpu.sync_copy(cur, acc_sh.at[ids_v.at[_perm(i)]], add=True)

    pltpu.async_copy(_feat_chunk(0), rows_v.at[0], sems.at[0])

    def chunk(i, carry):
        @pl.when(i % 2 == 0)
        def _():
            _step(i, rows_v.at[0], sems.at[0], rows_v.at[1], sems.at[1])

        @pl.when(i % 2 == 1)
        def _():
            _step(i, rows_v.at[1], sems.at[1], rows_v.at[0], sems.at[0])

        return carry

    lax.fori_loop(0, CPT, chunk, 0)
    plsc.subcore_barrier()

    g0 = s * SEG_PT
    pltpu.sync_copy(acc_sh.at[pl.ds(g0, SEG_PT)], facc_v)
    pltpu.sync_copy(facc_v, sums_hbm.at[c, pl.ds(g0, SEG_PT)])


def _hist_body(ids_ref, acc_ref):
    gcol = lax.broadcasted_iota(jnp.int32, (G, 1), 0)

    def blk(r, acc):
        rows = ids_ref[pl.ds(r * HB, HB), :]
        for q in range(HB):
            acc = acc + (gcol == rows[q:q + 1, :]).astype(jnp.float32)
        return acc

    acc_ref[...] = lax.fori_loop(0, IDR // HB, blk,
                                 jnp.zeros((G, D), jnp.float32))


_hist = pl.pallas_call(
    _hist_body,
    out_shape=jax.ShapeDtypeStruct((G, D), jnp.float32),
)


def _finalize_body(sums_ref, acc_ref, out_ref):
    total = sums_ref[0] + sums_ref[1]
    cnt = jnp.sum(acc_ref[...], axis=1, keepdims=True)
    out_ref[...] = total / jnp.maximum(cnt, 1.0)


_finalize = pl.pallas_call(
    _finalize_body,
    out_shape=jax.ShapeDtypeStruct((G, D), jnp.float32),
)


def kernel(feat, segment_ids, num_graphs):
    del num_graphs
    ids = segment_ids.astype(jnp.int32)
    sums = _seg_sum(feat, ids.reshape(NW, CPT, CH))
    acc = _hist(ids.reshape(IDR, D))
    return _finalize(sums, acc)
